# Initial kernel scaffold; baseline (speedup 1.0000x reference)
import jax
import jax.numpy as jnp
from jax.experimental import pallas as pl


def kernel(x, a, b, c, d, bias, w):
    raise NotImplementedError("write your pallas kernel here")


# speedup vs baseline: 1.0048x; 1.0048x over previous
import functools

import jax
import jax.numpy as jnp
from jax import lax
from jax.experimental import pallas as pl
from jax.experimental.pallas import tpu as pltpu

_C0, _C1, _C2 = 0.04, 5.0, 140.0
_THRESH = 30.0
_V_RESET = -65.0


def _snn_body(x_ref, p_ref, w_ref, out_ref, *, num_layers, steps, unroll):
    L = num_layers

    a_l = [p_ref[0, i] for i in range(L)]
    b_l = [p_ref[1, i] for i in range(L)]
    c_l = [p_ref[2, i] for i in range(L)]
    d_l = [p_ref[3, i] for i in range(L)]
    bias140_l = [p_ref[4, i] + _C2 for i in range(L)]

    v0 = tuple(jnp.full_like(b_l[i], _V_RESET) for i in range(L))
    u0 = tuple(b_l[i] * _V_RESET for i in range(L))

    def step(t, carry):
        v_st, u_st = carry
        v_st, u_st = list(v_st), list(u_st)
        s = x_ref[t]
        for i in range(L):
            if i > 0:
                s = jnp.sum(s[:, :, None] * w_ref[i - 1], axis=1)
            vi, ui = v_st[i], u_st[i]
            dv = (_C0 * vi + _C1) * vi + bias140_l[i] - ui + s
            du = a_l[i] * (b_l[i] * vi - ui)
            v_new = vi + dv
            u_new = ui + du
            spiked = v_new > _THRESH
            s = spiked.astype(jnp.float32)
            v_st[i] = jnp.where(spiked, c_l[i], v_new)
            u_st[i] = jnp.where(spiked, u_new + d_l[i], u_new)
        out_ref[t] = s
        return tuple(v_st), tuple(u_st)

    lax.fori_loop(0, steps, step, (v0, u0), unroll=unroll)


def _snn_forward(x, a, b, c, d, bias, w, *, steps, batch_block):
    T, B, N = x.shape
    L = a.shape[0]
    x = x[:steps]
    num_bblocks = B // batch_block

    params = jnp.stack([a, b, c, d, bias], axis=0)

    body = functools.partial(_snn_body, num_layers=L, steps=steps, unroll=16)

    out = pl.pallas_call(
        body,
        out_shape=jax.ShapeDtypeStruct((steps, B, N), jnp.float32),
        grid_spec=pltpu.PrefetchScalarGridSpec(
            num_scalar_prefetch=0,
            grid=(num_bblocks,),
            in_specs=[
                pl.BlockSpec((steps, batch_block, N), lambda g: (0, g, 0)),
                pl.BlockSpec((5, L, batch_block, N), lambda g: (0, 0, g, 0)),
                pl.BlockSpec((L - 1, batch_block, N, N),
                             lambda g: (0, g, 0, 0)),
            ],
            out_specs=pl.BlockSpec((steps, batch_block, N),
                                   lambda g: (0, g, 0)),
        ),
        compiler_params=pltpu.CompilerParams(
            dimension_semantics=("parallel",),
            vmem_limit_bytes=64 * 1024 * 1024,
        ),
    )(x, params, w)
    return out


def kernel(x, a, b, c, d, bias, w):
    return _snn_forward(x, a, b, c, d, bias, w, steps=320, batch_block=32)


# speedup vs baseline: 1.3532x; 1.3467x over previous
import functools

import jax
import jax.numpy as jnp
from jax import lax
from jax.experimental import pallas as pl
from jax.experimental.pallas import tpu as pltpu

_C0, _C1, _C2 = 0.04, 5.0, 140.0
_THRESH = 30.0
_V_RESET = -65.0


def _snn_body(x_ref, p_ref, w_ref, out_ref, *, num_layers, steps, unroll):
    L = num_layers
    N = x_ref.shape[2]

    a_l = [p_ref[0, i] for i in range(L)]
    b_l = [p_ref[1, i] for i in range(L)]
    c_l = [p_ref[2, i] for i in range(L)]
    d_l = [p_ref[3, i] for i in range(L)]
    bias140_l = [p_ref[4, i] + _C2 for i in range(L)]

    v0 = tuple(jnp.full_like(b_l[i], _V_RESET) for i in range(L))
    u0 = tuple(b_l[i] * _V_RESET for i in range(L))

    def matvec(layer, s):
        w2 = w_ref[layer]
        acc0 = s[:, 0:1] * w2[0]
        acc1 = s[:, 1:2] * w2[1]
        for n in range(2, N, 2):
            acc0 = acc0 + s[:, n:n + 1] * w2[n]
            acc1 = acc1 + s[:, n + 1:n + 2] * w2[n + 1]
        return acc0 + acc1

    def step(t, carry):
        v_st, u_st = carry
        v_st, u_st = list(v_st), list(u_st)
        s = x_ref[t]
        for i in range(L):
            if i > 0:
                s = matvec(i - 1, s)
            vi, ui = v_st[i], u_st[i]
            dv = (_C0 * vi + _C1) * vi + bias140_l[i] - ui + s
            du = a_l[i] * (b_l[i] * vi - ui)
            v_new = vi + dv
            u_new = ui + du
            spiked = v_new > _THRESH
            s = spiked.astype(jnp.float32)
            v_st[i] = jnp.where(spiked, c_l[i], v_new)
            u_st[i] = jnp.where(spiked, u_new + d_l[i], u_new)
        out_ref[t] = s
        return tuple(v_st), tuple(u_st)

    lax.fori_loop(0, steps, step, (v0, u0), unroll=unroll)


def _snn_forward(x, a, b, c, d, bias, w, *, steps, unroll):
    T, B, N = x.shape
    L = a.shape[0]
    x = x[:steps]

    params = jnp.stack([a, b, c, d, bias], axis=0)
    w2 = jnp.transpose(w, (0, 2, 1, 3))

    body = functools.partial(_snn_body, num_layers=L, steps=steps,
                             unroll=unroll)

    out = pl.pallas_call(
        body,
        out_shape=jax.ShapeDtypeStruct((steps, B, N), jnp.float32),
        grid_spec=pltpu.PrefetchScalarGridSpec(
            num_scalar_prefetch=0,
            grid=(1,),
            in_specs=[
                pl.BlockSpec((steps, B, N), lambda g: (0, 0, 0)),
                pl.BlockSpec((5, L, B, N), lambda g: (0, 0, 0, 0)),
                pl.BlockSpec((L - 1, N, B, N), lambda g: (0, 0, 0, 0)),
            ],
            out_specs=pl.BlockSpec((steps, B, N), lambda g: (0, 0, 0)),
        ),
        compiler_params=pltpu.CompilerParams(
            dimension_semantics=("arbitrary",),
            vmem_limit_bytes=64 * 1024 * 1024,
        ),
    )(x, params, w2)
    return out


def kernel(x, a, b, c, d, bias, w):
    return _snn_forward(x, a, b, c, d, bias, w, steps=320, unroll=16)
